# Initial kernel scaffold; baseline (speedup 1.0000x reference)
import jax
import jax.numpy as jnp
from jax.experimental import pallas as pl


def kernel(items_emb, user_emb, target_items):
    raise NotImplementedError("write your pallas kernel here")


# speedup vs baseline: 1.7563x; 1.7563x over previous
import functools

import jax
import jax.numpy as jnp
from jax import lax
from jax.experimental import pallas as pl
from jax.experimental.pallas import tpu as pltpu

N, D, T = 100000, 32, 8
B = 4096
NB = -(-N // B)
NP = NB * B
RB = B // 128
R = NP // 128
NEG = -1e30
BIGI = 2**31 - 1


def _body(tgt_sm, items_blk, u_ref, items_any, out_ref, scr, w, sem):
    k = pl.program_id(0)

    @pl.when(k == 0)
    def _init():
        w[...] = jnp.zeros((16, D), jnp.float32)
        w[0:1, :] = u_ref[...]
        copies = []
        for i in range(T):
            c = pltpu.make_async_copy(
                items_any.at[pl.ds(tgt_sm[i], 1), :],
                w.at[pl.ds(1 + i, 1), :],
                sem,
            )
            c.start()
            copies.append(c)
        for c in copies:
            c.wait()

    x = items_blk[...]
    s = lax.dot_general(w[...], x, (((1,), (1,)), ((), ())),
                        preferred_element_type=jnp.float32)
    scr[:, pl.ds(k * RB, RB), :] = s.reshape(16, RB, 128)

    @pl.when(k == NB - 1)
    def _select():
        gidx = (lax.broadcasted_iota(jnp.int32, (R, 128), 0) * 128
                + lax.broadcasted_iota(jnp.int32, (R, 128), 1))
        valid = gidx < N
        s0 = scr[0]
        sm = jnp.where(valid, s0, NEG)

        top_v, top_i = [], []
        for _ in range(6):
            m = jnp.max(sm)
            sel = jnp.min(jnp.where(sm == m, gidx, BIGI))
            top_v.append(m)
            top_i.append(sel)
            sm = jnp.where(gidx == sel, NEG, sm)

        loss = jnp.float32(0.0)
        for t in range(T):
            tt = tgt_sm[t]
            s_t = jnp.sum(w[0, :] * w[1 + t, :])

            in_first5 = jnp.zeros((), jnp.bool_)
            for i in range(5):
                in_first5 = in_first5 | (top_i[i] == tt)
            contrib = jnp.float32(0.0)
            for i in range(5):
                contrib += jnp.where(top_i[i] == tt, 0.0,
                                     jax.nn.sigmoid(top_v[i] - s_t))
            contrib += jnp.where(in_first5,
                                 jax.nn.sigmoid(top_v[5] - s_t), 0.0)

            sv = jnp.where(valid, scr[1 + t], NEG)
            sv = jnp.where(gidx == tt, NEG, sv)
            for i in range(5):
                sv = jnp.where(gidx == top_i[i], NEG, sv)
            sv = jnp.where(in_first5 & (gidx == top_i[5]), NEG, sv)
            for _ in range(5):
                m = jnp.max(sv)
                sel = jnp.min(jnp.where(sv == m, gidx, BIGI))
                sc = jnp.sum(jnp.where(gidx == sel, s0, 0.0))
                contrib += jax.nn.sigmoid(sc - s_t)
                sv = jnp.where(gidx == sel, NEG, sv)

            loss += contrib
        out_ref[...] = jnp.broadcast_to(loss, (1, 1))


def kernel(items_emb, user_emb, target_items):
    items_pad = jnp.pad(items_emb, ((0, NP - N), (0, 0)))
    grid_spec = pltpu.PrefetchScalarGridSpec(
        num_scalar_prefetch=1,
        grid=(NB,),
        in_specs=[
            pl.BlockSpec((B, D), lambda k, tgt: (k, 0)),
            pl.BlockSpec((1, D), lambda k, tgt: (0, 0)),
            pl.BlockSpec(memory_space=pltpu.MemorySpace.HBM),
        ],
        out_specs=pl.BlockSpec((1, 1), lambda k, tgt: (0, 0)),
        scratch_shapes=[
            pltpu.VMEM((16, R, 128), jnp.float32),
            pltpu.VMEM((16, D), jnp.float32),
            pltpu.SemaphoreType.DMA,
        ],
    )
    out = pl.pallas_call(
        _body,
        grid_spec=grid_spec,
        out_shape=jax.ShapeDtypeStruct((1, 1), jnp.float32),
    )(target_items, items_pad, user_emb, items_emb)
    return out[0, 0]
